# Initial kernel scaffold; baseline (speedup 1.0000x reference)
import jax
import jax.numpy as jnp
from jax.experimental import pallas as pl


def kernel(x, token_table, pos_table, stream_emb):
    raise NotImplementedError("write your pallas kernel here")


# speedup vs baseline: 4.0497x; 4.0497x over previous
import functools

import jax
import jax.numpy as jnp
from jax import lax
from jax.experimental import pallas as pl
from jax.experimental.pallas import tpu as pltpu
from jax.experimental.pallas import tpu_sc as plsc

DIM = 128
LANES = 16
NUM_CORES = 2
NUM_SUBCORES = 16
NUM_WORKERS = NUM_CORES * NUM_SUBCORES


def _build_kernel(B, N, S, V):
    assert S == 2 and DIM == 128
    assert B % NUM_WORKERS == 0
    b_per_w = B // NUM_WORKERS
    chunks = []
    off = 0
    while off < N:
        c = min(128, N - off)
        chunks.append((off, c))
        off += c

    mesh = plsc.VectorSubcoreMesh(core_axis_name="c", subcore_axis_name="s")

    @functools.partial(
        pl.kernel,
        mesh=mesh,
        out_type=jax.ShapeDtypeStruct((B * S * N, DIM), jnp.float32),
        scratch_types=[
            pltpu.VMEM((N,), jnp.int32),
            pltpu.VMEM((N, DIM), jnp.float32),
            pltpu.VMEM((N, DIM), jnp.float32),
            pltpu.VMEM((N, DIM), jnp.float32),
            pltpu.VMEM((S, DIM), jnp.float32),
            pltpu.SemaphoreType.DMA,
        ],
    )
    def k(x_hbm, table_hbm, pos_hbm, stream_hbm, out_hbm,
          idx_v, rows_v, t2_v, pos_v, stream_v, sem):
        wid = lax.axis_index("s") * NUM_CORES + lax.axis_index("c")
        base_b = wid * b_per_w

        pltpu.sync_copy(pos_hbm.at[pl.ds(0, N)], pos_v)
        pltpu.sync_copy(stream_hbm, stream_v)

        s0 = [stream_v[0, pl.ds(l * LANES, LANES)] for l in range(DIM // LANES)]
        s1 = [stream_v[1, pl.ds(l * LANES, LANES)] for l in range(DIM // LANES)]
        d = [s1[l] - s0[l] for l in range(DIM // LANES)]

        def body_b(i, carry):
            b = base_b + i
            idx_off = pl.multiple_of(b * N, 8)
            pltpu.sync_copy(x_hbm.at[pl.ds(idx_off, N)], idx_v)
            for (coff, clen) in chunks:
                pltpu.async_copy(
                    table_hbm.at[idx_v.at[pl.ds(coff, clen)]],
                    rows_v.at[pl.ds(coff, clen)],
                    sem,
                ).wait()

            def body_n(n, carry_n):
                for l in range(DIM // LANES):
                    sl = pl.ds(l * LANES, LANES)
                    t0 = rows_v[n, sl] + pos_v[n, sl] + s0[l]
                    rows_v[n, sl] = t0
                    t2_v[n, sl] = t0 + d[l]
                return carry_n

            lax.fori_loop(0, N, body_n, 0)

            out_off0 = pl.multiple_of(b * (S * N), 8)
            out_off1 = pl.multiple_of(b * (S * N) + N, 8)
            pltpu.sync_copy(rows_v, out_hbm.at[pl.ds(out_off0, N)])
            pltpu.sync_copy(t2_v, out_hbm.at[pl.ds(out_off1, N)])
            return carry

        lax.fori_loop(0, b_per_w, body_b, 0)

    return k


def kernel(x, token_table, pos_table, stream_emb):
    B, N = x.shape
    S, D = stream_emb.shape
    V = token_table.shape[0]
    xflat = x.reshape(B * N).astype(jnp.int32)
    k = _build_kernel(B, N, S, V)
    out = k(xflat, token_table, pos_table, stream_emb)
    return out.reshape(B * S, N, D)
